# baseline (speedup 1.0000x reference)
import functools

import jax
import jax.numpy as jnp
from jax import lax
from jax.experimental import pallas as pl
from jax.experimental.pallas import tpu as pltpu, tpu_sc as plsc

B, S, D = 1, 2048, 2048
H = D // 2
E = 8
K = 2
R = 8
DO = 2048
EPS = 1e-5
ER = E * R
TT = 512
T = B * S

_HI = jax.lax.Precision.HIGHEST
_DEF = jax.lax.Precision.DEFAULT

_NC = 2
_NS = 16
_NW = _NC * _NS
_TPW = T // _NW
_CPW = _TPW // 16


def _dot(a, b, prec=_DEF):
    return jnp.dot(a, b, precision=prec, preferred_element_type=jnp.float32)


def _gate_kernel(x_ref, w1_ref, b1_ref, w2_ref, b2_ref, a2d_ref, gain_ref,
                 bias_ref, logits_ref, hg_ref):
    xt = x_ref[...]
    h = jax.nn.gelu(_dot(xt, w1_ref[...]) + b1_ref[...])
    logits = _dot(h, w2_ref[...]) + b2_ref[...]
    logits_ref[...] = logits.T
    ha = _dot(xt, a2d_ref[...])
    ii = jax.lax.broadcasted_iota(jnp.int32, (ER, ER), 0)
    jj = jax.lax.broadcasted_iota(jnp.int32, (ER, ER), 1)
    avg = jnp.where(ii // R == jj // R, 1.0 / R, 0.0)
    mu = _dot(ha, avg)
    dev = ha - mu
    var = _dot(dev * dev, avg)
    hn = dev * jax.lax.rsqrt(var + EPS)
    hg_ref[...] = hn * gain_ref[...] + bias_ref[...]


def _cf(v):
    return jnp.full((16,), v, jnp.float32)


def _ci(v):
    return jnp.full((16,), v, jnp.int32)


def _route_sc_kernel(logits_hbm, gates_hbm, lg_v, gf_v):
    wid = lax.axis_index("s") * _NC + lax.axis_index("c")
    base = wid * _TPW
    for e in range(E):
        pltpu.sync_copy(logits_hbm.at[e, pl.ds(base, _TPW)], lg_v.at[e])
    for c in range(_CPW):
        le = [lg_v[e, pl.ds(c * 16, 16)] for e in range(E)]
        v1 = le[0]
        i1 = _ci(0)
        for e in range(1, E):
            b = le[e] > v1
            i1 = jnp.where(b, _ci(e), i1)
            v1 = jnp.where(b, le[e], v1)
        neg = _cf(-1e30)
        v2 = neg
        i2 = _ci(E)
        for e in range(E):
            cand = jnp.where(i1 == _ci(e), neg, le[e])
            b = cand > v2
            i2 = jnp.where(b, _ci(e), i2)
            v2 = jnp.where(b, cand, v2)
        e2 = jnp.exp(v2 - v1)
        g1 = _cf(1.0) / (_cf(1.0) + e2)
        g2 = e2 * g1
        zero = _cf(0.0)
        for e in range(E):
            ge = jnp.where(i1 == _ci(e), g1, jnp.where(i2 == _ci(e), g2, zero))
            gf_v[e, pl.ds(c * 16, 16)] = ge
    for e in range(E):
        pltpu.sync_copy(gf_v.at[e], gates_hbm.at[e, pl.ds(base, _TPW)])


_route_sc = functools.partial(
    pl.kernel,
    out_type=jax.ShapeDtypeStruct((E, T), jnp.float32),
    mesh=plsc.VectorSubcoreMesh(core_axis_name="c", subcore_axis_name="s"),
    scratch_types=[
        pltpu.VMEM((E, _TPW), jnp.float32),
        pltpu.VMEM((E, _TPW), jnp.float32),
    ],
)(_route_sc_kernel)


def _combine_kernel(hg_ref, gates_ref, b2d_ref, out_ref):
    ei = jax.lax.broadcasted_iota(jnp.int32, (E, ER), 0)
    ej = jax.lax.broadcasted_iota(jnp.int32, (E, ER), 1)
    expand = (ei == ej // R).astype(jnp.float32)
    gexp = _dot(gates_ref[...].T, expand)
    out_ref[...] = _dot(hg_ref[...] * gexp, b2d_ref[...])


@functools.partial(jax.jit, static_argnames=())
def kernel(x, W1, b1, W2, b2, A, Bm, scaling, ln_g, ln_b):
    xf = x.reshape(T, D)
    a2d = jnp.transpose(A, (1, 0, 2)).reshape(D, ER)
    b2d = Bm.reshape(ER, DO)
    gain = (ln_g * scaling[:, None]).reshape(1, ER)
    bias = (ln_b * scaling[:, None]).reshape(1, ER)
    b1r = b1.reshape(1, H)
    b2r = b2.reshape(1, E)

    grid = (T // TT,)
    logits, hg = pl.pallas_call(
        _gate_kernel,
        grid=grid,
        in_specs=[
            pl.BlockSpec((TT, D), lambda i: (i, 0)),
            pl.BlockSpec((D, H), lambda i: (0, 0)),
            pl.BlockSpec((1, H), lambda i: (0, 0)),
            pl.BlockSpec((H, E), lambda i: (0, 0)),
            pl.BlockSpec((1, E), lambda i: (0, 0)),
            pl.BlockSpec((D, ER), lambda i: (0, 0)),
            pl.BlockSpec((1, ER), lambda i: (0, 0)),
            pl.BlockSpec((1, ER), lambda i: (0, 0)),
        ],
        out_specs=[
            pl.BlockSpec((E, TT), lambda i: (0, i)),
            pl.BlockSpec((TT, ER), lambda i: (i, 0)),
        ],
        out_shape=[
            jax.ShapeDtypeStruct((E, T), jnp.float32),
            jax.ShapeDtypeStruct((T, ER), jnp.float32),
        ],
    )(xf, W1, b1r, W2, b2r, a2d, gain, bias)

    gates = _route_sc(logits)

    out = pl.pallas_call(
        _combine_kernel,
        grid=grid,
        in_specs=[
            pl.BlockSpec((TT, ER), lambda i: (i, 0)),
            pl.BlockSpec((E, TT), lambda i: (0, i)),
            pl.BlockSpec((ER, DO), lambda i: (0, 0)),
        ],
        out_specs=pl.BlockSpec((TT, DO), lambda i: (i, 0)),
        out_shape=jax.ShapeDtypeStruct((T, DO), jnp.float32),
    )(hg, gates, b2d)
    return out.reshape(B, S, DO)


# speedup vs baseline: 1.4032x; 1.4032x over previous
import functools

import jax
import jax.numpy as jnp
from jax.experimental import pallas as pl

B, S, D = 1, 2048, 2048
H = D // 2
E = 8
K = 2
R = 8
DO = 2048
EPS = 1e-5
ER = E * R
TT = 256

_HI = jax.lax.Precision.HIGHEST
_DEF = jax.lax.Precision.DEFAULT


def _dot(a, b, prec=_DEF):
    return jnp.dot(a, b, precision=prec, preferred_element_type=jnp.float32)


def _fused_kernel(x_ref, w1_ref, b1_ref, w2_ref, b2_ref, a2d_ref, gain_ref,
                  bias_ref, b2d_ref, out_ref):
    xt = x_ref[...]
    h = jax.nn.gelu(_dot(xt, w1_ref[...]) + b1_ref[...])
    logits = _dot(h, w2_ref[...]) + b2_ref[...]
    lb = jax.lax.bitcast_convert_type(logits, jnp.int32)
    mono = lb ^ jax.lax.shift_right_arithmetic(lb, 31) & 0x7FFFFFFF
    idx = jax.lax.broadcasted_iota(jnp.int32, (TT, E), 1)
    key = (mono & ~7) + (7 - idx)
    k1 = jnp.max(key, axis=1, keepdims=True)
    sel1 = key == k1
    k2 = jnp.max(jnp.where(sel1, jnp.iinfo(jnp.int32).min, key),
                 axis=1, keepdims=True)
    sel2 = key == k2

    def _decode(k):
        m = k & ~7
        return jax.lax.bitcast_convert_type(
            m ^ jax.lax.shift_right_arithmetic(m, 31) & 0x7FFFFFFF,
            jnp.float32)

    e2 = jnp.exp(_decode(k2) - _decode(k1))
    g1 = 1.0 / (1.0 + e2)
    g2 = e2 * g1
    gfull = jnp.where(sel1, g1, 0.0) + jnp.where(sel2, g2, 0.0)
    ei = jax.lax.broadcasted_iota(jnp.int32, (E, ER), 0)
    ej = jax.lax.broadcasted_iota(jnp.int32, (E, ER), 1)
    expand = (ei == ej // R).astype(jnp.float32)
    gexp = _dot(gfull, expand)
    ha = _dot(xt, a2d_ref[...])
    ii = jax.lax.broadcasted_iota(jnp.int32, (ER, ER), 0)
    jj = jax.lax.broadcasted_iota(jnp.int32, (ER, ER), 1)
    avg = jnp.where(ii // R == jj // R, 1.0 / R, 0.0)
    mu = _dot(ha, avg)
    dev = ha - mu
    var = _dot(dev * dev, avg)
    hn = dev * jax.lax.rsqrt(var + EPS)
    hc = (hn * gain_ref[...] + bias_ref[...]) * gexp
    out_ref[...] = _dot(hc, b2d_ref[...])


@functools.partial(jax.jit, static_argnames=())
def kernel(x, W1, b1, W2, b2, A, Bm, scaling, ln_g, ln_b):
    T = B * S
    xf = x.reshape(T, D)
    a2d = jnp.transpose(A, (1, 0, 2)).reshape(D, ER)
    b2d = Bm.reshape(ER, DO)
    gain = (ln_g * scaling[:, None]).reshape(1, ER)
    bias = (ln_b * scaling[:, None]).reshape(1, ER)
    b1r = b1.reshape(1, H)
    b2r = b2.reshape(1, E)

    grid = (T // TT,)
    out = pl.pallas_call(
        _fused_kernel,
        grid=grid,
        in_specs=[
            pl.BlockSpec((TT, D), lambda i: (i, 0)),
            pl.BlockSpec((D, H), lambda i: (0, 0)),
            pl.BlockSpec((1, H), lambda i: (0, 0)),
            pl.BlockSpec((H, E), lambda i: (0, 0)),
            pl.BlockSpec((1, E), lambda i: (0, 0)),
            pl.BlockSpec((D, ER), lambda i: (0, 0)),
            pl.BlockSpec((1, ER), lambda i: (0, 0)),
            pl.BlockSpec((1, ER), lambda i: (0, 0)),
            pl.BlockSpec((ER, DO), lambda i: (0, 0)),
        ],
        out_specs=pl.BlockSpec((TT, DO), lambda i: (i, 0)),
        out_shape=jax.ShapeDtypeStruct((T, DO), jnp.float32),
    )(xf, W1, b1r, W2, b2r, a2d, gain, bias, b2d)
    return out.reshape(B, S, DO)


# speedup vs baseline: 1.5552x; 1.1083x over previous
import functools

import jax
import jax.numpy as jnp
from jax.experimental import pallas as pl

B, S, D = 1, 2048, 2048
H = D // 2
E = 8
K = 2
R = 8
DO = 2048
EPS = 1e-5
ER = E * R
TT = 512

_HI = jax.lax.Precision.HIGHEST
_DEF = jax.lax.Precision.DEFAULT


def _dot(a, b, prec=_DEF):
    return jnp.dot(a, b, precision=prec, preferred_element_type=jnp.float32)


def _fused_kernel(x_ref, w1_ref, b1_ref, w2_ref, b2_ref, a2d_ref, gain_ref,
                  bias_ref, b2d_ref, out_ref):
    xt = x_ref[...]
    h = jax.nn.gelu(_dot(xt, w1_ref[...]) + b1_ref[...])
    logits = _dot(h, w2_ref[...]) + b2_ref[...]
    lb = jax.lax.bitcast_convert_type(logits, jnp.int32)
    mono = lb ^ jax.lax.shift_right_arithmetic(lb, 31) & 0x7FFFFFFF
    idx = jax.lax.broadcasted_iota(jnp.int32, (TT, E), 1)
    key = (mono & ~7) + (7 - idx)
    k1 = jnp.max(key, axis=1, keepdims=True)
    sel1 = key == k1
    k2 = jnp.max(jnp.where(sel1, jnp.iinfo(jnp.int32).min, key),
                 axis=1, keepdims=True)
    sel2 = key == k2

    def _decode(k):
        m = k & ~7
        return jax.lax.bitcast_convert_type(
            m ^ jax.lax.shift_right_arithmetic(m, 31) & 0x7FFFFFFF,
            jnp.float32)

    e2 = jnp.exp(_decode(k2) - _decode(k1))
    g1 = 1.0 / (1.0 + e2)
    g2 = e2 * g1
    gfull = jnp.where(sel1, g1, 0.0) + jnp.where(sel2, g2, 0.0)
    ei = jax.lax.broadcasted_iota(jnp.int32, (E, ER), 0)
    ej = jax.lax.broadcasted_iota(jnp.int32, (E, ER), 1)
    expand = (ei == ej // R).astype(jnp.float32)
    gexp = _dot(gfull, expand)
    ha = _dot(xt, a2d_ref[...])
    ii = jax.lax.broadcasted_iota(jnp.int32, (ER, ER), 0)
    jj = jax.lax.broadcasted_iota(jnp.int32, (ER, ER), 1)
    avg = jnp.where(ii // R == jj // R, 1.0 / R, 0.0)
    mu = _dot(ha, avg)
    dev = ha - mu
    var = _dot(dev * dev, avg)
    hn = dev * jax.lax.rsqrt(var + EPS)
    hc = (hn * gain_ref[...] + bias_ref[...]) * gexp
    out_ref[...] = _dot(hc, b2d_ref[...])


@functools.partial(jax.jit, static_argnames=())
def kernel(x, W1, b1, W2, b2, A, Bm, scaling, ln_g, ln_b):
    T = B * S
    xf = x.reshape(T, D)
    a2d = jnp.transpose(A, (1, 0, 2)).reshape(D, ER)
    b2d = Bm.reshape(ER, DO)
    gain = (ln_g * scaling[:, None]).reshape(1, ER)
    bias = (ln_b * scaling[:, None]).reshape(1, ER)
    b1r = b1.reshape(1, H)
    b2r = b2.reshape(1, E)

    grid = (T // TT,)
    out = pl.pallas_call(
        _fused_kernel,
        grid=grid,
        in_specs=[
            pl.BlockSpec((TT, D), lambda i: (i, 0)),
            pl.BlockSpec((D, H), lambda i: (0, 0)),
            pl.BlockSpec((1, H), lambda i: (0, 0)),
            pl.BlockSpec((H, E), lambda i: (0, 0)),
            pl.BlockSpec((1, E), lambda i: (0, 0)),
            pl.BlockSpec((D, ER), lambda i: (0, 0)),
            pl.BlockSpec((1, ER), lambda i: (0, 0)),
            pl.BlockSpec((1, ER), lambda i: (0, 0)),
            pl.BlockSpec((ER, DO), lambda i: (0, 0)),
        ],
        out_specs=pl.BlockSpec((TT, DO), lambda i: (i, 0)),
        out_shape=jax.ShapeDtypeStruct((T, DO), jnp.float32),
    )(xf, W1, b1r, W2, b2r, a2d, gain, bias, b2d)
    return out.reshape(B, S, DO)
